# baseline (speedup 1.0000x reference)
import jax
import jax.numpy as jnp
from jax.experimental import pallas as pl

_N_RAYS = 4096
_N_SAMPLES = 192
_NEAR, _FAR = 2.0, 6.0
_STEP = (_FAR - _NEAR) / _N_SAMPLES
_LOG_EPS = -23.025850929940457
_BLOCK_R = 256
_N_CHAINS = 2


def _render_rays(rays, w6, b6, wcat, bs, br, tri, t_mid2):
    f32 = jnp.float32
    r = rays.shape[0]
    ab = jnp.dot(rays, w6, preferred_element_type=f32) + b6
    c = ab.reshape(r, 2, 128)
    ones = jnp.ones((1, _N_SAMPLES), dtype=f32)
    tmat = jnp.concatenate([ones[:, None, :], t_mid2[:, None, :]], axis=1)
    tmat = jnp.broadcast_to(tmat, (r, 2, _N_SAMPLES))
    h = jax.lax.dot_general(
        c, tmat, dimension_numbers=(((1,), (1,)), ((0,), (0,))),
        preferred_element_type=f32)
    h = jnp.maximum(h, 0.0)

    wcat_b = jnp.broadcast_to(wcat[None], (r, 4, 128))
    z = jax.lax.dot_general(
        wcat_b, h, dimension_numbers=(((2,), (1,)), ((0,), (0,))),
        preferred_element_type=f32)

    sigma = jax.nn.softplus(z[:, 0, :] + bs)
    x = sigma * _STEP
    alpha = 1.0 - jnp.exp(-x)
    log_trans = jnp.maximum(-x, _LOG_EPS)
    excl = jnp.dot(log_trans, tri, precision=jax.lax.Precision.HIGHEST,
                   preferred_element_type=f32)
    weights = alpha * jnp.exp(excl)

    outs = []
    for ch in range(3):
        rgb_c = jax.nn.sigmoid(z[:, 1 + ch, :] + br[ch])
        outs.append(jnp.sum(weights * rgb_c, axis=-1)[:, None])
    rgb = jnp.concatenate(outs, axis=1)
    op = jnp.sum(weights, axis=-1)[:, None]
    depth = jnp.sum(weights * t_mid2, axis=-1)[:, None]
    return rgb, op, depth


def _render_block(rays_ref, w6_ref, b6_ref, wcat_ref, bs_ref, br_ref, tri_ref,
                  rgb_ref, op_ref, depth_ref):
    f32 = jnp.float32
    w6 = w6_ref[...]
    b6 = b6_ref[...]
    wcat = wcat_ref[...]
    tri = tri_ref[...]
    bs = bs_ref[0, 0]
    br = (br_ref[0, 0], br_ref[0, 1], br_ref[0, 2])
    s_idx = jax.lax.broadcasted_iota(jnp.int32, (1, _N_SAMPLES), 1).astype(f32)
    t_mid2 = _NEAR + (s_idx + 0.5) * _STEP

    half = _BLOCK_R // _N_CHAINS
    for i in range(_N_CHAINS):
        rows = pl.ds(i * half, half)
        rays = rays_ref[rows, :] * (2.0 / 3.0)
        rgb, op, depth = _render_rays(rays, w6, b6, wcat, bs, br, tri, t_mid2)
        rgb_ref[rows, :] = rgb
        op_ref[rows, :] = op
        depth_ref[rows, :] = depth


@jax.jit
def kernel(rays, W1, b1, W_sigma, b_sigma, W_rgb, b_rgb):
    n_rays = rays.shape[0]
    wcat = jnp.concatenate([W_sigma, W_rgb], axis=1).T
    zero3 = jnp.zeros((3, 128), dtype=jnp.float32)
    w6 = jnp.concatenate(
        [jnp.concatenate([W1, zero3], axis=1),
         jnp.concatenate([zero3, W1], axis=1)], axis=0)
    b6 = jnp.concatenate([b1, jnp.zeros_like(b1)]).reshape(1, 256)
    bs_2d = b_sigma.reshape(1, 1)
    br_2d = b_rgb.reshape(1, 3)
    s = _N_SAMPLES
    tri = (jnp.arange(s, dtype=jnp.int32)[:, None]
           < jnp.arange(s, dtype=jnp.int32)[None, :]).astype(jnp.float32)
    grid = (n_rays // _BLOCK_R,)
    rgb, op, depth = pl.pallas_call(
        _render_block,
        grid=grid,
        in_specs=[
            pl.BlockSpec((_BLOCK_R, 6), lambda i: (i, 0)),
            pl.BlockSpec((6, 256), lambda i: (0, 0)),
            pl.BlockSpec((1, 256), lambda i: (0, 0)),
            pl.BlockSpec((4, 128), lambda i: (0, 0)),
            pl.BlockSpec((1, 1), lambda i: (0, 0)),
            pl.BlockSpec((1, 3), lambda i: (0, 0)),
            pl.BlockSpec((s, s), lambda i: (0, 0)),
        ],
        out_specs=[
            pl.BlockSpec((_BLOCK_R, 3), lambda i: (i, 0)),
            pl.BlockSpec((_BLOCK_R, 1), lambda i: (i, 0)),
            pl.BlockSpec((_BLOCK_R, 1), lambda i: (i, 0)),
        ],
        out_shape=[
            jax.ShapeDtypeStruct((n_rays, 3), jnp.float32),
            jax.ShapeDtypeStruct((n_rays, 1), jnp.float32),
            jax.ShapeDtypeStruct((n_rays, 1), jnp.float32),
        ],
    )(rays, w6, b6, wcat, bs_2d, br_2d, tri)
    return rgb, op[:, 0], depth[:, 0]


# speedup vs baseline: 1.0533x; 1.0533x over previous
import jax
import jax.numpy as jnp
from jax.experimental import pallas as pl

_N_RAYS = 4096
_N_SAMPLES = 192
_NEAR, _FAR = 2.0, 6.0
_STEP = (_FAR - _NEAR) / _N_SAMPLES
_LOG_EPS = -23.025850929940457
_BLOCK_R = 256
_N_CHAINS = 2


def _render_rays(rays, w6, b6, wcat, bs, br, tri, t_mid2):
    f32 = jnp.float32
    r = rays.shape[0]
    ab = jnp.dot(rays, w6, preferred_element_type=f32) + b6
    c = ab.reshape(r, 2, 128)
    ones = jnp.ones((1, _N_SAMPLES), dtype=f32)
    tmat = jnp.concatenate([ones[:, None, :], t_mid2[:, None, :]], axis=1)
    tmat = jnp.broadcast_to(tmat, (r, 2, _N_SAMPLES))
    h = jax.lax.dot_general(
        c, tmat, dimension_numbers=(((1,), (1,)), ((0,), (0,))),
        preferred_element_type=f32)
    h = jnp.maximum(h, 0.0)

    wcat_b = jnp.broadcast_to(wcat[None], (r, 4, 128))
    z = jax.lax.dot_general(
        wcat_b, h, dimension_numbers=(((2,), (1,)), ((0,), (0,))),
        preferred_element_type=f32)

    sigma = jax.nn.softplus(z[:, 0, :] + bs)
    x = sigma * _STEP
    alpha = 1.0 - jnp.exp(-x)
    log_trans = jnp.maximum(-x, _LOG_EPS)
    excl = jnp.dot(log_trans, tri, preferred_element_type=f32)
    weights = alpha * jnp.exp(excl)

    outs = []
    for ch in range(3):
        rgb_c = jax.nn.sigmoid(z[:, 1 + ch, :] + br[ch])
        outs.append(jnp.sum(weights * rgb_c, axis=-1)[:, None])
    rgb = jnp.concatenate(outs, axis=1)
    op = jnp.sum(weights, axis=-1)[:, None]
    depth = jnp.sum(weights * t_mid2, axis=-1)[:, None]
    return rgb, op, depth


def _render_block(rays_ref, w6_ref, b6_ref, wcat_ref, bs_ref, br_ref, tri_ref,
                  rgb_ref, op_ref, depth_ref):
    f32 = jnp.float32
    w6 = w6_ref[...]
    b6 = b6_ref[...]
    wcat = wcat_ref[...]
    tri = tri_ref[...]
    bs = bs_ref[0, 0]
    br = (br_ref[0, 0], br_ref[0, 1], br_ref[0, 2])
    s_idx = jax.lax.broadcasted_iota(jnp.int32, (1, _N_SAMPLES), 1).astype(f32)
    t_mid2 = _NEAR + (s_idx + 0.5) * _STEP

    half = _BLOCK_R // _N_CHAINS
    for i in range(_N_CHAINS):
        rows = pl.ds(i * half, half)
        rays = rays_ref[rows, :] * (2.0 / 3.0)
        rgb, op, depth = _render_rays(rays, w6, b6, wcat, bs, br, tri, t_mid2)
        rgb_ref[rows, :] = rgb
        op_ref[rows, :] = op
        depth_ref[rows, :] = depth


@jax.jit
def kernel(rays, W1, b1, W_sigma, b_sigma, W_rgb, b_rgb):
    n_rays = rays.shape[0]
    wcat = jnp.concatenate([W_sigma, W_rgb], axis=1).T
    zero3 = jnp.zeros((3, 128), dtype=jnp.float32)
    w6 = jnp.concatenate(
        [jnp.concatenate([W1, zero3], axis=1),
         jnp.concatenate([zero3, W1], axis=1)], axis=0)
    b6 = jnp.concatenate([b1, jnp.zeros_like(b1)]).reshape(1, 256)
    bs_2d = b_sigma.reshape(1, 1)
    br_2d = b_rgb.reshape(1, 3)
    s = _N_SAMPLES
    tri = (jnp.arange(s, dtype=jnp.int32)[:, None]
           < jnp.arange(s, dtype=jnp.int32)[None, :]).astype(jnp.float32)
    grid = (n_rays // _BLOCK_R,)
    rgb, op, depth = pl.pallas_call(
        _render_block,
        grid=grid,
        in_specs=[
            pl.BlockSpec((_BLOCK_R, 6), lambda i: (i, 0)),
            pl.BlockSpec((6, 256), lambda i: (0, 0)),
            pl.BlockSpec((1, 256), lambda i: (0, 0)),
            pl.BlockSpec((4, 128), lambda i: (0, 0)),
            pl.BlockSpec((1, 1), lambda i: (0, 0)),
            pl.BlockSpec((1, 3), lambda i: (0, 0)),
            pl.BlockSpec((s, s), lambda i: (0, 0)),
        ],
        out_specs=[
            pl.BlockSpec((_BLOCK_R, 3), lambda i: (i, 0)),
            pl.BlockSpec((_BLOCK_R, 1), lambda i: (i, 0)),
            pl.BlockSpec((_BLOCK_R, 1), lambda i: (i, 0)),
        ],
        out_shape=[
            jax.ShapeDtypeStruct((n_rays, 3), jnp.float32),
            jax.ShapeDtypeStruct((n_rays, 1), jnp.float32),
            jax.ShapeDtypeStruct((n_rays, 1), jnp.float32),
        ],
    )(rays, w6, b6, wcat, bs_2d, br_2d, tri)
    return rgb, op[:, 0], depth[:, 0]


# speedup vs baseline: 1.0735x; 1.0192x over previous
import jax
import jax.numpy as jnp
from jax.experimental import pallas as pl

_N_RAYS = 4096
_N_SAMPLES = 192
_NEAR, _FAR = 2.0, 6.0
_STEP = (_FAR - _NEAR) / _N_SAMPLES
_LOG_EPS = -23.025850929940457
_BLOCK_R = 512
_N_CHAINS = 2


def _render_rays(rays, w6, b6, wcat, bs, br, tri, t_mid2):
    f32 = jnp.float32
    r = rays.shape[0]
    ab = jnp.dot(rays, w6, preferred_element_type=f32) + b6
    c = ab.reshape(r, 2, 128)
    ones = jnp.ones((1, _N_SAMPLES), dtype=f32)
    tmat = jnp.concatenate([ones[:, None, :], t_mid2[:, None, :]], axis=1)
    tmat = jnp.broadcast_to(tmat, (r, 2, _N_SAMPLES))
    h = jax.lax.dot_general(
        c, tmat, dimension_numbers=(((1,), (1,)), ((0,), (0,))),
        preferred_element_type=f32)
    h = jnp.maximum(h, 0.0)

    wcat_b = jnp.broadcast_to(wcat[None], (r, 4, 128))
    z = jax.lax.dot_general(
        wcat_b, h, dimension_numbers=(((2,), (1,)), ((0,), (0,))),
        preferred_element_type=f32)

    sigma = jax.nn.softplus(z[:, 0, :] + bs)
    x = sigma * _STEP
    alpha = 1.0 - jnp.exp(-x)
    log_trans = jnp.maximum(-x, _LOG_EPS)
    excl = jnp.dot(log_trans, tri, preferred_element_type=f32)
    weights = alpha * jnp.exp(excl)

    outs = []
    for ch in range(3):
        rgb_c = jax.nn.sigmoid(z[:, 1 + ch, :] + br[ch])
        outs.append(jnp.sum(weights * rgb_c, axis=-1)[:, None])
    rgb = jnp.concatenate(outs, axis=1)
    op = jnp.sum(weights, axis=-1)[:, None]
    depth = jnp.sum(weights * t_mid2, axis=-1)[:, None]
    return rgb, op, depth


def _render_block(rays_ref, w6_ref, b6_ref, wcat_ref, bs_ref, br_ref, tri_ref,
                  rgb_ref, op_ref, depth_ref):
    f32 = jnp.float32
    w6 = w6_ref[...]
    b6 = b6_ref[...]
    wcat = wcat_ref[...]
    tri = tri_ref[...]
    bs = bs_ref[0, 0]
    br = (br_ref[0, 0], br_ref[0, 1], br_ref[0, 2])
    s_idx = jax.lax.broadcasted_iota(jnp.int32, (1, _N_SAMPLES), 1).astype(f32)
    t_mid2 = _NEAR + (s_idx + 0.5) * _STEP

    half = _BLOCK_R // _N_CHAINS
    for i in range(_N_CHAINS):
        rows = pl.ds(i * half, half)
        rays = rays_ref[rows, :] * (2.0 / 3.0)
        rgb, op, depth = _render_rays(rays, w6, b6, wcat, bs, br, tri, t_mid2)
        rgb_ref[rows, :] = rgb
        op_ref[rows, :] = op
        depth_ref[rows, :] = depth


@jax.jit
def kernel(rays, W1, b1, W_sigma, b_sigma, W_rgb, b_rgb):
    n_rays = rays.shape[0]
    wcat = jnp.concatenate([W_sigma, W_rgb], axis=1).T
    zero3 = jnp.zeros((3, 128), dtype=jnp.float32)
    w6 = jnp.concatenate(
        [jnp.concatenate([W1, zero3], axis=1),
         jnp.concatenate([zero3, W1], axis=1)], axis=0)
    b6 = jnp.concatenate([b1, jnp.zeros_like(b1)]).reshape(1, 256)
    bs_2d = b_sigma.reshape(1, 1)
    br_2d = b_rgb.reshape(1, 3)
    s = _N_SAMPLES
    tri = (jnp.arange(s, dtype=jnp.int32)[:, None]
           < jnp.arange(s, dtype=jnp.int32)[None, :]).astype(jnp.float32)
    grid = (n_rays // _BLOCK_R,)
    rgb, op, depth = pl.pallas_call(
        _render_block,
        grid=grid,
        in_specs=[
            pl.BlockSpec((_BLOCK_R, 6), lambda i: (i, 0)),
            pl.BlockSpec((6, 256), lambda i: (0, 0)),
            pl.BlockSpec((1, 256), lambda i: (0, 0)),
            pl.BlockSpec((4, 128), lambda i: (0, 0)),
            pl.BlockSpec((1, 1), lambda i: (0, 0)),
            pl.BlockSpec((1, 3), lambda i: (0, 0)),
            pl.BlockSpec((s, s), lambda i: (0, 0)),
        ],
        out_specs=[
            pl.BlockSpec((_BLOCK_R, 3), lambda i: (i, 0)),
            pl.BlockSpec((_BLOCK_R, 1), lambda i: (i, 0)),
            pl.BlockSpec((_BLOCK_R, 1), lambda i: (i, 0)),
        ],
        out_shape=[
            jax.ShapeDtypeStruct((n_rays, 3), jnp.float32),
            jax.ShapeDtypeStruct((n_rays, 1), jnp.float32),
            jax.ShapeDtypeStruct((n_rays, 1), jnp.float32),
        ],
    )(rays, w6, b6, wcat, bs_2d, br_2d, tri)
    return rgb, op[:, 0], depth[:, 0]
